# baseline (speedup 1.0000x reference)
import jax
import jax.numpy as jnp
import numpy as np
from jax import lax
from jax.experimental import pallas as pl
from jax.experimental.pallas import tpu as pltpu
from jax.experimental.pallas import tpu_sc as plsc

X, Y, H = 256, 256, 16
N_COARSE = 16384
BT = 8
NC, NS = 2, 16
NW = NC * NS
PER_W = N_COARSE // NW
CHUNK = 128
NCHUNK = PER_W // CHUNK
SPAN = PER_W * H

SCALE_XY = 102.4
DIST_XY = 51.2
SCALE_H = 8.0
DIST_H = 5.0

_GDN = lax.GatherDimensionNumbers(
    offset_dims=(), collapsed_slice_dims=(0,), start_index_map=(0,))


def _splat(vec, k):
    idx = jnp.full((16, 1), k, jnp.int32)
    return lax.gather(vec, idx, dimension_numbers=_GDN, slice_sizes=(1,),
                      mode=lax.GatherScatterMode.PROMISE_IN_BOUNDS)


def _body(table, rnd3, btzf, btzi, coords_out, idx_out,
          idx_v, xidx_v, yidx_v, rows_x, rows_y, cbuf, ibuf,
          btzf_v, btzi_v, gsem, osem):
    wid = lax.axis_index("s") * NC + lax.axis_index("c")
    pltpu.sync_copy(rnd3.at[wid], idx_v)
    pltpu.sync_copy(btzf, btzf_v)
    pltpu.sync_copy(btzi, btzi_v)
    for k in range(NCHUNK):
        for j in range(CHUNK // 16):
            sl = pl.ds(j * 16, 16)
            two_i = idx_v[k, sl] * 2
            xidx_v[k, sl] = two_i
            yidx_v[k, sl] = two_i + 1
    gathers = []
    for k in range(NCHUNK):
        gathers.append(pltpu.async_copy(
            table.at[xidx_v.at[k]], rows_x.at[pl.ds(k * CHUNK, CHUNK)], gsem))
        gathers.append(pltpu.async_copy(
            table.at[yidx_v.at[k]], rows_y.at[pl.ds(k * CHUNK, CHUNK)], gsem))

    vf = btzf_v[...]
    vi = btzi_v[...]
    lanes = lax.iota(jnp.int32, 16)
    hcoord = lanes.astype(jnp.float32) * (1.0 / (H - 1)) * SCALE_H - DIST_H + vf
    hidx = lanes + vi
    for g in gathers:
        g.wait()

    def body(i, carry):
        xv = rows_x[pl.ds(i * 16, 16)]
        yv = rows_y[pl.ds(i * 16, 16)]
        for k in range(16):
            xs = _splat(xv, k)
            ys = _splat(yv, k)
            sl = pl.ds((i * 16 + k) * H, H)
            cbuf[0, sl] = xs * SCALE_XY - DIST_XY + vf
            cbuf[1, sl] = ys * SCALE_XY - DIST_XY + vf
            cbuf[2, sl] = hcoord
            ibuf[0, sl] = (xs * float(X - 1) + 0.5).astype(jnp.int32) + vi
            ibuf[1, sl] = (ys * float(Y - 1) + 0.5).astype(jnp.int32) + vi
            ibuf[2, sl] = hidx
        return carry

    lax.fori_loop(0, PER_W // 16, body, 0)

    base = wid * SPAN
    copies = []
    for b in range(BT):
        copies.append(pltpu.async_copy(
            cbuf, coords_out.at[b, :, pl.ds(base, SPAN)], osem))
        copies.append(pltpu.async_copy(
            ibuf, idx_out.at[b, :, pl.ds(base, SPAN)], osem))
    for c in copies:
        c.wait()


def kernel(grid, rnd, bt):
    table = grid.reshape(X * Y * 2)
    rnd3 = rnd.reshape(NW, NCHUNK, CHUNK)
    btz = (jnp.asarray(bt) - BT).astype(jnp.int32)
    btzi = jnp.full((16,), btz, jnp.int32)
    btzf = btzi.astype(jnp.float32)

    mesh = plsc.VectorSubcoreMesh(
        core_axis_name="c", subcore_axis_name="s",
        num_cores=NC, num_subcores=NS)
    run = pl.kernel(
        _body,
        out_type=(
            jax.ShapeDtypeStruct((BT, 3, N_COARSE * H), jnp.float32),
            jax.ShapeDtypeStruct((BT, 3, N_COARSE * H), jnp.int32),
        ),
        mesh=mesh,
        scratch_types=[
            pltpu.VMEM((NCHUNK, CHUNK), jnp.int32),
            pltpu.VMEM((NCHUNK, CHUNK), jnp.int32),
            pltpu.VMEM((NCHUNK, CHUNK), jnp.int32),
            pltpu.VMEM((PER_W,), jnp.float32),
            pltpu.VMEM((PER_W,), jnp.float32),
            pltpu.VMEM((3, SPAN), jnp.float32),
            pltpu.VMEM((3, SPAN), jnp.int32),
            pltpu.VMEM((16,), jnp.float32),
            pltpu.VMEM((16,), jnp.int32),
            pltpu.SemaphoreType.DMA,
            pltpu.SemaphoreType.DMA,
        ],
    )
    return run(table, rnd3, btzf, btzi)


# speedup vs baseline: 1.0143x; 1.0143x over previous
import jax
import jax.numpy as jnp
from jax import lax
from jax.experimental import pallas as pl
from jax.experimental.pallas import tpu as pltpu
from jax.experimental.pallas import tpu_sc as plsc

X, Y, H = 256, 256, 16
N_COARSE = 16384
BT = 8
NC, NS = 2, 16
NW = NC * NS
PER_W = N_COARSE // NW
CHUNK = 128
NCHUNK = PER_W // CHUNK
SPAN = PER_W * H
NJ = N_COARSE * H

SCALE_XY = 102.4
DIST_XY = 51.2
SCALE_H = 8.0
DIST_H = 5.0

_GDN = lax.GatherDimensionNumbers(
    offset_dims=(), collapsed_slice_dims=(0,), start_index_map=(0,))


def _splat(vec, k):
    idx = jnp.full((16, 1), k, jnp.int32)
    return lax.gather(vec, idx, dimension_numbers=_GDN, slice_sizes=(1,),
                      mode=lax.GatherScatterMode.PROMISE_IN_BOUNDS)


def _sc_body(table, rnd, xexp_out, yexp_out,
             idx_v, xidx_v, yidx_v, rows_x, rows_y, xe, ye, gsem, osem):
    wid = lax.axis_index("s") * NC + lax.axis_index("c")
    base = wid * PER_W
    pltpu.sync_copy(rnd.at[pl.ds(base, PER_W)], idx_v)
    for k in range(NCHUNK):
        for j in range(CHUNK // 16):
            sl = pl.ds(k * CHUNK + j * 16, 16)
            two_i = idx_v[sl] * 2
            xidx_v[k, pl.ds(j * 16, 16)] = two_i
            yidx_v[k, pl.ds(j * 16, 16)] = two_i + 1
    gathers = []
    for k in range(NCHUNK):
        gathers.append(pltpu.async_copy(
            table.at[xidx_v.at[k]], rows_x.at[pl.ds(k * CHUNK, CHUNK)], gsem))
        gathers.append(pltpu.async_copy(
            table.at[yidx_v.at[k]], rows_y.at[pl.ds(k * CHUNK, CHUNK)], gsem))
    for g in gathers:
        g.wait()

    def body(i, carry):
        xv = rows_x[pl.ds(i * 16, 16)]
        yv = rows_y[pl.ds(i * 16, 16)]
        for k in range(16):
            sl = pl.ds((i * 16 + k) * H, H)
            xe[sl] = _splat(xv, k)
            ye[sl] = _splat(yv, k)
        return carry

    lax.fori_loop(0, PER_W // 16, body, 0)

    c1 = pltpu.async_copy(xe, xexp_out.at[0, pl.ds(base * H, SPAN)], osem)
    c2 = pltpu.async_copy(ye, yexp_out.at[0, pl.ds(base * H, SPAN)], osem)
    c1.wait()
    c2.wait()


TCW = NJ // 8


def _tc_body(xexp_ref, yexp_ref, btzf_ref, btzi_ref, coords_ref, idx_ref):
    btzf = btzf_ref[0, 0]
    btzi = btzi_ref[0, 0]
    xv = xexp_ref[...]
    yv = yexp_ref[...]
    h = lax.broadcasted_iota(jnp.int32, (1, TCW), 1) & (H - 1)
    hf = h.astype(jnp.float32) * (1.0 / (H - 1))
    cx = xv * SCALE_XY - DIST_XY + btzf
    cy = yv * SCALE_XY - DIST_XY + btzf
    ch = hf * SCALE_H - DIST_H + btzf
    cat_c = jnp.concatenate([cx, cy, ch], axis=0)
    ix = (xv * float(X - 1) + 0.5).astype(jnp.int32) + btzi
    iy = (yv * float(Y - 1) + 0.5).astype(jnp.int32) + btzi
    ih = h + btzi
    cat_i = jnp.concatenate([ix, iy, ih], axis=0)
    for b in range(BT):
        coords_ref[b] = cat_c
        idx_ref[b] = cat_i


def kernel(grid, rnd, bt):
    table = grid.reshape(X * Y * 2)
    btz = (jnp.asarray(bt) - BT).astype(jnp.int32)
    btzi = btz.reshape(1, 1)
    btzf = btzi.astype(jnp.float32)

    mesh = plsc.VectorSubcoreMesh(
        core_axis_name="c", subcore_axis_name="s",
        num_cores=NC, num_subcores=NS)
    sc_run = pl.kernel(
        _sc_body,
        out_type=(
            jax.ShapeDtypeStruct((1, NJ), jnp.float32),
            jax.ShapeDtypeStruct((1, NJ), jnp.float32),
        ),
        mesh=mesh,
        scratch_types=[
            pltpu.VMEM((PER_W,), jnp.int32),
            pltpu.VMEM((NCHUNK, CHUNK), jnp.int32),
            pltpu.VMEM((NCHUNK, CHUNK), jnp.int32),
            pltpu.VMEM((PER_W,), jnp.float32),
            pltpu.VMEM((PER_W,), jnp.float32),
            pltpu.VMEM((SPAN,), jnp.float32),
            pltpu.VMEM((SPAN,), jnp.float32),
            pltpu.SemaphoreType.DMA,
            pltpu.SemaphoreType.DMA,
        ],
    )
    xexp, yexp = sc_run(table, rnd)

    coords, vidx = pl.pallas_call(
        _tc_body,
        grid=(NJ // TCW,),
        in_specs=[
            pl.BlockSpec((1, TCW), lambda n: (0, n)),
            pl.BlockSpec((1, TCW), lambda n: (0, n)),
            pl.BlockSpec(memory_space=pltpu.SMEM),
            pl.BlockSpec(memory_space=pltpu.SMEM),
        ],
        out_specs=[
            pl.BlockSpec((BT, 3, TCW), lambda n: (0, 0, n)),
            pl.BlockSpec((BT, 3, TCW), lambda n: (0, 0, n)),
        ],
        out_shape=(
            jax.ShapeDtypeStruct((BT, 3, NJ), jnp.float32),
            jax.ShapeDtypeStruct((BT, 3, NJ), jnp.int32),
        ),
    )(xexp, yexp, btzf, btzi)
    return (coords, vidx)


# speedup vs baseline: 1.7015x; 1.6776x over previous
import jax
import jax.numpy as jnp
from jax import lax
from jax.experimental import pallas as pl
from jax.experimental.pallas import tpu as pltpu
from jax.experimental.pallas import tpu_sc as plsc

X, Y, H = 256, 256, 16
N_COARSE = 16384
BT = 8
NC, NS = 2, 16
NW = NC * NS
PER_W = N_COARSE // NW
CHUNK = 128
NCHUNK = PER_W // CHUNK
SPAN = PER_W * H
NJ = N_COARSE * H

SCALE_XY = 102.4
DIST_XY = 51.2
SCALE_H = 8.0
DIST_H = 5.0

_GDN = lax.GatherDimensionNumbers(
    offset_dims=(), collapsed_slice_dims=(0,), start_index_map=(0,))


def _splat(vec, k):
    idx = jnp.full((16, 1), k, jnp.int32)
    return lax.gather(vec, idx, dimension_numbers=_GDN, slice_sizes=(1,),
                      mode=lax.GatherScatterMode.PROMISE_IN_BOUNDS)


def _sc_body(table, rnd, xexp_out, yexp_out,
             idx_v, xidx_v, yidx_v, rows_x, rows_y, xe, ye, gsem, osem):
    wid = lax.axis_index("s") * NC + lax.axis_index("c")
    base = wid * PER_W
    pltpu.sync_copy(rnd.at[pl.ds(base, PER_W)], idx_v)
    for k in range(NCHUNK):
        for j in range(CHUNK // 16):
            sl = pl.ds(k * CHUNK + j * 16, 16)
            two_i = idx_v[sl] * 2
            xidx_v[k, pl.ds(j * 16, 16)] = two_i
            yidx_v[k, pl.ds(j * 16, 16)] = two_i + 1
    gathers = []
    for k in range(NCHUNK):
        gathers.append(pltpu.async_copy(
            table.at[xidx_v.at[k]], rows_x.at[pl.ds(k * CHUNK, CHUNK)], gsem))
        gathers.append(pltpu.async_copy(
            table.at[yidx_v.at[k]], rows_y.at[pl.ds(k * CHUNK, CHUNK)], gsem))
    for g in gathers:
        g.wait()

    def body(i, carry):
        xv = rows_x[pl.ds(i * 16, 16)]
        yv = rows_y[pl.ds(i * 16, 16)]
        for k in range(16):
            sl = pl.ds((i * 16 + k) * H, H)
            xe[sl] = _splat(xv, k)
            ye[sl] = _splat(yv, k)
        return carry

    lax.fori_loop(0, PER_W // 16, body, 0)

    c1 = pltpu.async_copy(xe, xexp_out.at[0, pl.ds(base * H, SPAN)], osem)
    c2 = pltpu.async_copy(ye, yexp_out.at[0, pl.ds(base * H, SPAN)], osem)
    c1.wait()
    c2.wait()


TCW = NJ // 8


def _tc_body(xexp_ref, yexp_ref, btzf_ref, btzi_ref, coords_ref, idx_ref):
    btzf = btzf_ref[0, 0]
    btzi = btzi_ref[0, 0]
    xv = xexp_ref[...]
    yv = yexp_ref[...]
    h = lax.broadcasted_iota(jnp.int32, (1, TCW), 1) & (H - 1)
    hf = h.astype(jnp.float32) * (1.0 / (H - 1))
    cx = xv * SCALE_XY - DIST_XY + btzf
    cy = yv * SCALE_XY - DIST_XY + btzf
    ch = hf * SCALE_H - DIST_H + btzf
    ix = (xv * float(X - 1) + 0.5).astype(jnp.int32) + btzi
    iy = (yv * float(Y - 1) + 0.5).astype(jnp.int32) + btzi
    ih = h + btzi
    coords_ref[0] = jnp.broadcast_to(cx, (BT, TCW))
    coords_ref[1] = jnp.broadcast_to(cy, (BT, TCW))
    coords_ref[2] = jnp.broadcast_to(ch, (BT, TCW))
    idx_ref[0] = jnp.broadcast_to(ix, (BT, TCW))
    idx_ref[1] = jnp.broadcast_to(iy, (BT, TCW))
    idx_ref[2] = jnp.broadcast_to(ih, (BT, TCW))


def kernel(grid, rnd, bt):
    table = grid.reshape(X * Y * 2)
    btz = (jnp.asarray(bt) - BT).astype(jnp.int32)
    btzi = btz.reshape(1, 1)
    btzf = btzi.astype(jnp.float32)

    mesh = plsc.VectorSubcoreMesh(
        core_axis_name="c", subcore_axis_name="s",
        num_cores=NC, num_subcores=NS)
    sc_run = pl.kernel(
        _sc_body,
        out_type=(
            jax.ShapeDtypeStruct((1, NJ), jnp.float32),
            jax.ShapeDtypeStruct((1, NJ), jnp.float32),
        ),
        mesh=mesh,
        scratch_types=[
            pltpu.VMEM((PER_W,), jnp.int32),
            pltpu.VMEM((NCHUNK, CHUNK), jnp.int32),
            pltpu.VMEM((NCHUNK, CHUNK), jnp.int32),
            pltpu.VMEM((PER_W,), jnp.float32),
            pltpu.VMEM((PER_W,), jnp.float32),
            pltpu.VMEM((SPAN,), jnp.float32),
            pltpu.VMEM((SPAN,), jnp.float32),
            pltpu.SemaphoreType.DMA,
            pltpu.SemaphoreType.DMA,
        ],
    )
    xexp, yexp = sc_run(table, rnd)

    coords, vidx = pl.pallas_call(
        _tc_body,
        grid=(NJ // TCW,),
        in_specs=[
            pl.BlockSpec((1, TCW), lambda n: (0, n)),
            pl.BlockSpec((1, TCW), lambda n: (0, n)),
            pl.BlockSpec(memory_space=pltpu.SMEM),
            pl.BlockSpec(memory_space=pltpu.SMEM),
        ],
        out_specs=[
            pl.BlockSpec((3, BT, TCW), lambda n: (0, 0, n)),
            pl.BlockSpec((3, BT, TCW), lambda n: (0, 0, n)),
        ],
        out_shape=(
            jax.ShapeDtypeStruct((3, BT, NJ), jnp.float32),
            jax.ShapeDtypeStruct((3, BT, NJ), jnp.int32),
        ),
    )(xexp, yexp, btzf, btzi)
    return (coords.transpose(1, 0, 2), vidx.transpose(1, 0, 2))


# speedup vs baseline: 3.2020x; 1.8819x over previous
import jax
import jax.numpy as jnp
from jax import lax
from jax.experimental import pallas as pl
from jax.experimental.pallas import tpu as pltpu
from jax.experimental.pallas import tpu_sc as plsc

X, Y, H = 256, 256, 16
N_COARSE = 16384
BT = 8
NC, NS = 2, 16
NW = NC * NS
PER_W = N_COARSE // NW
CHUNK = 128
NCHUNK = PER_W // CHUNK
SPAN = PER_W * H
NJ = N_COARSE * H

SCALE_XY = 102.4
DIST_XY = 51.2
SCALE_H = 8.0
DIST_H = 5.0

_GDN = lax.GatherDimensionNumbers(
    offset_dims=(), collapsed_slice_dims=(0,), start_index_map=(0,))


def _splat(vec, k):
    idx = jnp.full((16, 1), k, jnp.int32)
    return lax.gather(vec, idx, dimension_numbers=_GDN, slice_sizes=(1,),
                      mode=lax.GatherScatterMode.PROMISE_IN_BOUNDS)


def _sc_body(xtab, ytab, rnd, xexp_out, yexp_out,
             idx_v, rows_x, rows_y, xe, ye, gsem, osem):
    wid = lax.axis_index("s") * NC + lax.axis_index("c")
    base = wid * PER_W
    pltpu.sync_copy(rnd.at[pl.ds(base, PER_W)], idx_v)
    gathers = []
    for k in range(NCHUNK):
        sl = pl.ds(k * CHUNK, CHUNK)
        gathers.append(pltpu.async_copy(
            xtab.at[idx_v.at[sl]], rows_x.at[sl], gsem))
        gathers.append(pltpu.async_copy(
            ytab.at[idx_v.at[sl]], rows_y.at[sl], gsem))
    for g in gathers:
        g.wait()

    def body(i, carry):
        xv = rows_x[pl.ds(i * 16, 16)]
        yv = rows_y[pl.ds(i * 16, 16)]
        for k in range(16):
            sl = pl.ds((i * 16 + k) * H, H)
            xe[sl] = _splat(xv, k)
            ye[sl] = _splat(yv, k)
        return carry

    lax.fori_loop(0, PER_W // 16, body, 0)

    c1 = pltpu.async_copy(xe, xexp_out.at[0, pl.ds(base * H, SPAN)], osem)
    c2 = pltpu.async_copy(ye, yexp_out.at[0, pl.ds(base * H, SPAN)], osem)
    c1.wait()
    c2.wait()


TCW = NJ // 8


def _tc_body(xexp_ref, yexp_ref, btzf_ref, btzi_ref, coords_ref, idx_ref):
    btzf = btzf_ref[0, 0]
    btzi = btzi_ref[0, 0]
    xv = xexp_ref[...]
    yv = yexp_ref[...]
    h = lax.broadcasted_iota(jnp.int32, (1, TCW), 1) & (H - 1)
    hf = h.astype(jnp.float32) * (1.0 / (H - 1))
    cx = xv * SCALE_XY - DIST_XY + btzf
    cy = yv * SCALE_XY - DIST_XY + btzf
    ch = hf * SCALE_H - DIST_H + btzf
    ix = (xv * float(X - 1) + 0.5).astype(jnp.int32) + btzi
    iy = (yv * float(Y - 1) + 0.5).astype(jnp.int32) + btzi
    ih = h + btzi
    coords_ref[0] = jnp.broadcast_to(cx, (BT, TCW))
    coords_ref[1] = jnp.broadcast_to(cy, (BT, TCW))
    coords_ref[2] = jnp.broadcast_to(ch, (BT, TCW))
    idx_ref[0] = jnp.broadcast_to(ix, (BT, TCW))
    idx_ref[1] = jnp.broadcast_to(iy, (BT, TCW))
    idx_ref[2] = jnp.broadcast_to(ih, (BT, TCW))


def kernel(grid, rnd, bt):
    xtab = grid[:, :, 0].reshape(X * Y)
    ytab = grid[:, :, 1].reshape(X * Y)
    btz = (jnp.asarray(bt) - BT).astype(jnp.int32)
    btzi = btz.reshape(1, 1)
    btzf = btzi.astype(jnp.float32)

    mesh = plsc.VectorSubcoreMesh(
        core_axis_name="c", subcore_axis_name="s",
        num_cores=NC, num_subcores=NS)
    sc_run = pl.kernel(
        _sc_body,
        out_type=(
            jax.ShapeDtypeStruct((1, NJ), jnp.float32),
            jax.ShapeDtypeStruct((1, NJ), jnp.float32),
        ),
        mesh=mesh,
        scratch_types=[
            pltpu.VMEM((PER_W,), jnp.int32),
            pltpu.VMEM((PER_W,), jnp.float32),
            pltpu.VMEM((PER_W,), jnp.float32),
            pltpu.VMEM((SPAN,), jnp.float32),
            pltpu.VMEM((SPAN,), jnp.float32),
            pltpu.SemaphoreType.DMA,
            pltpu.SemaphoreType.DMA,
        ],
    )
    xexp, yexp = sc_run(xtab, ytab, rnd)

    coords, vidx = pl.pallas_call(
        _tc_body,
        grid=(NJ // TCW,),
        in_specs=[
            pl.BlockSpec((1, TCW), lambda n: (0, n)),
            pl.BlockSpec((1, TCW), lambda n: (0, n)),
            pl.BlockSpec(memory_space=pltpu.SMEM),
            pl.BlockSpec(memory_space=pltpu.SMEM),
        ],
        out_specs=[
            pl.BlockSpec((3, BT, TCW), lambda n: (0, 0, n)),
            pl.BlockSpec((3, BT, TCW), lambda n: (0, 0, n)),
        ],
        out_shape=(
            jax.ShapeDtypeStruct((3, BT, NJ), jnp.float32),
            jax.ShapeDtypeStruct((3, BT, NJ), jnp.int32),
        ),
    )(xexp, yexp, btzf, btzi)
    return (coords.transpose(1, 0, 2), vidx.transpose(1, 0, 2))


# speedup vs baseline: 3.4412x; 1.0747x over previous
import jax
import jax.numpy as jnp
from jax import lax
from jax.experimental import pallas as pl
from jax.experimental.pallas import tpu as pltpu
from jax.experimental.pallas import tpu_sc as plsc

X, Y, H = 256, 256, 16
N_COARSE = 16384
BT = 8
NC, NS = 2, 16
NW = NC * NS
PER_W = N_COARSE // NW
CHUNK = 128
NCHUNK = PER_W // CHUNK
SPAN = PER_W * H
NJ = N_COARSE * H

SCALE_XY = 102.4
DIST_XY = 51.2
SCALE_H = 8.0
DIST_H = 5.0

_GDN = lax.GatherDimensionNumbers(
    offset_dims=(), collapsed_slice_dims=(0,), start_index_map=(0,))


def _splat(vec, k):
    idx = jnp.full((16, 1), k, jnp.int32)
    return lax.gather(vec, idx, dimension_numbers=_GDN, slice_sizes=(1,),
                      mode=lax.GatherScatterMode.PROMISE_IN_BOUNDS)


def _sc_body(xtab, ytab, rnd, xexp_out, yexp_out,
             idx_v, rows_x, rows_y, xe, ye, gsem, osem):
    wid = lax.axis_index("s") * NC + lax.axis_index("c")
    base = wid * PER_W
    pltpu.sync_copy(rnd.at[pl.ds(base, PER_W)], idx_v)
    gathers = []
    for k in range(NCHUNK):
        sl = pl.ds(k * CHUNK, CHUNK)
        gathers.append(pltpu.async_copy(
            xtab.at[idx_v.at[sl]], rows_x.at[sl], gsem))
        gathers.append(pltpu.async_copy(
            ytab.at[idx_v.at[sl]], rows_y.at[sl], gsem))
    for g in gathers:
        g.wait()

    def body(i, carry):
        xv = rows_x[pl.ds(i * 16, 16)]
        yv = rows_y[pl.ds(i * 16, 16)]
        for k in range(16):
            sl = pl.ds((i * 16 + k) * H, H)
            xe[sl] = _splat(xv, k)
            ye[sl] = _splat(yv, k)
        return carry

    lax.fori_loop(0, PER_W // 16, body, 0)

    c1 = pltpu.async_copy(xe, xexp_out.at[0, pl.ds(base * H, SPAN)], osem)
    c2 = pltpu.async_copy(ye, yexp_out.at[0, pl.ds(base * H, SPAN)], osem)
    c1.wait()
    c2.wait()


TCW = NJ // 8


def _tc_heights(btzf_ref, btzi_ref, coords_ref, idx_ref):
    btzf = btzf_ref[0, 0]
    btzi = btzi_ref[0, 0]
    h = lax.broadcasted_iota(jnp.int32, (1, TCW), 1) & (H - 1)
    hf = h.astype(jnp.float32) * (1.0 / (H - 1))
    ch = hf * SCALE_H - DIST_H + btzf
    ih = h + btzi
    coords_ref[0] = jnp.broadcast_to(ch, (BT, TCW))
    idx_ref[0] = jnp.broadcast_to(ih, (BT, TCW))


def _tc_xy(xexp_ref, yexp_ref, btzf_ref, btzi_ref, c_in_ref, i_in_ref,
           coords_ref, idx_ref):
    del c_in_ref, i_in_ref
    btzf = btzf_ref[0, 0]
    btzi = btzi_ref[0, 0]
    xv = xexp_ref[...]
    yv = yexp_ref[...]
    cx = xv * SCALE_XY - DIST_XY + btzf
    cy = yv * SCALE_XY - DIST_XY + btzf
    ix = (xv * float(X - 1) + 0.5).astype(jnp.int32) + btzi
    iy = (yv * float(Y - 1) + 0.5).astype(jnp.int32) + btzi
    coords_ref[0] = jnp.broadcast_to(cx, (BT, TCW))
    coords_ref[1] = jnp.broadcast_to(cy, (BT, TCW))
    idx_ref[0] = jnp.broadcast_to(ix, (BT, TCW))
    idx_ref[1] = jnp.broadcast_to(iy, (BT, TCW))


def kernel(grid, rnd, bt):
    xtab = grid[:, :, 0].reshape(X * Y)
    ytab = grid[:, :, 1].reshape(X * Y)
    btz = (jnp.asarray(bt) - BT).astype(jnp.int32)
    btzi = btz.reshape(1, 1)
    btzf = btzi.astype(jnp.float32)

    mesh = plsc.VectorSubcoreMesh(
        core_axis_name="c", subcore_axis_name="s",
        num_cores=NC, num_subcores=NS)
    sc_run = pl.kernel(
        _sc_body,
        out_type=(
            jax.ShapeDtypeStruct((1, NJ), jnp.float32),
            jax.ShapeDtypeStruct((1, NJ), jnp.float32),
        ),
        mesh=mesh,
        scratch_types=[
            pltpu.VMEM((PER_W,), jnp.int32),
            pltpu.VMEM((PER_W,), jnp.float32),
            pltpu.VMEM((PER_W,), jnp.float32),
            pltpu.VMEM((SPAN,), jnp.float32),
            pltpu.VMEM((SPAN,), jnp.float32),
            pltpu.SemaphoreType.DMA,
            pltpu.SemaphoreType.DMA,
        ],
    )
    xexp, yexp = sc_run(xtab, ytab, rnd)

    coords_h, vidx_h = pl.pallas_call(
        _tc_heights,
        grid=(NJ // TCW,),
        in_specs=[
            pl.BlockSpec(memory_space=pltpu.SMEM),
            pl.BlockSpec(memory_space=pltpu.SMEM),
        ],
        out_specs=[
            pl.BlockSpec((1, BT, TCW), lambda n: (2, 0, n)),
            pl.BlockSpec((1, BT, TCW), lambda n: (2, 0, n)),
        ],
        out_shape=(
            jax.ShapeDtypeStruct((3, BT, NJ), jnp.float32),
            jax.ShapeDtypeStruct((3, BT, NJ), jnp.int32),
        ),
    )(btzf, btzi)

    coords, vidx = pl.pallas_call(
        _tc_xy,
        grid=(NJ // TCW,),
        in_specs=[
            pl.BlockSpec((1, TCW), lambda n: (0, n)),
            pl.BlockSpec((1, TCW), lambda n: (0, n)),
            pl.BlockSpec(memory_space=pltpu.SMEM),
            pl.BlockSpec(memory_space=pltpu.SMEM),
            pl.BlockSpec(memory_space=pl.ANY),
            pl.BlockSpec(memory_space=pl.ANY),
        ],
        out_specs=[
            pl.BlockSpec((2, BT, TCW), lambda n: (0, 0, n)),
            pl.BlockSpec((2, BT, TCW), lambda n: (0, 0, n)),
        ],
        out_shape=(
            jax.ShapeDtypeStruct((3, BT, NJ), jnp.float32),
            jax.ShapeDtypeStruct((3, BT, NJ), jnp.int32),
        ),
        input_output_aliases={4: 0, 5: 1},
    )(xexp, yexp, btzf, btzi, coords_h, vidx_h)
    return (coords.transpose(1, 0, 2), vidx.transpose(1, 0, 2))
